# baseline (speedup 1.0000x reference)
import functools

import jax
import jax.numpy as jnp
from jax import lax
from jax.experimental import pallas as pl
from jax.experimental.pallas import tpu as pltpu
from jax.experimental.pallas import tpu_sc as plsc


@functools.lru_cache(maxsize=None)
def _make_sc_tgather(B, S, V, DP):
    info = plsc.get_sparse_core_info()
    NC, NS = info.num_cores, info.num_subcores
    NW = NC * NS
    L = 128
    NBT = B // L
    NVJ = DP // L
    NVT = V // 8
    VJ7 = (V - (NVJ - 1) * L) // 8
    n_tasks = S * NVJ * NBT
    assert n_tasks % NW == 0 and B % L == 0 and V % 8 == 0 and DP % L == 0
    t_per_w = n_tasks // NW
    mesh = plsc.VectorSubcoreMesh(core_axis_name="c", subcore_axis_name="s")

    @functools.partial(
        pl.kernel,
        mesh=mesh,
        compiler_params=pltpu.CompilerParams(
            use_tc_tiling_on_sc=True, needs_layout_passes=False),
        out_type=jax.ShapeDtypeStruct((S, NVT, NBT, 8, L), jnp.float32),
        scratch_types=(
            [pltpu.VMEM((t_per_w * L,), jnp.int32)]
            + [pltpu.VMEM((L, L), jnp.float32) for _ in range(2)]
            + [pltpu.VMEM((16, 8, L), jnp.float32) for _ in range(2)]
            + [pltpu.SemaphoreType.DMA, pltpu.SemaphoreType.DMA,
               pltpu.SemaphoreType.DMA, pltpu.SemaphoreType.DMA]
        ),
    )
    def tg_kernel(table8_hbm, idx_hbm, z_hbm, idx_v, src0, src1, dst0, dst1,
                  gs0, gs1, ssA, ssB):
        srcs = (src0, src1)
        dsts = (dst0, dst1)
        gsems = (gs0, gs1)
        wid = lax.axis_index("s") * NC + lax.axis_index("c")
        t0 = wid * t_per_w
        pltpu.sync_copy(idx_hbm.at[pl.ds(t0 * L, t_per_w * L)], idx_v)

        b_idx = [lax.iota(jnp.int32, 16) + 16 * jj for jj in range(8)]

        def start_gather(j, sl):
            pltpu.async_copy(
                table8_hbm.at[idx_v.at[pl.ds(j * L, L)]], srcs[sl], gsems[sl])

        def wait_gather(sl):
            pltpu.make_async_copy(
                table8_hbm.at[idx_v.at[pl.ds(0, L)]], srcs[sl],
                gsems[sl]).wait()

        def task_svb(j):
            t = t0 + j
            s = t // (NVJ * NBT)
            vj = (t // NBT) % NVJ
            bt = t % NBT
            return s, vj, bt

        def transpose(sl, nv):
            src, dst = srcs[sl], dsts[sl]

            def body(v, c):
                vt = v // 8
                vs = v % 8
                vsp = jnp.broadcast_to(v, (16,)).astype(jnp.int32)
                for jj in range(8):
                    vals = plsc.load_gather(src, [b_idx[jj], vsp])
                    dst[vt, vs, pl.ds(jj * 16, 16)] = vals
                return c

            lax.fori_loop(0, nv, body, 0)

        def start_scatter(j, sl):
            s, vj, bt = task_svb(j)

            @pl.when(vj < NVJ - 1)
            def _():
                pltpu.async_copy(
                    dsts[sl].at[pl.ds(0, 16)],
                    z_hbm.at[s, pl.ds(vj * 16, 16), bt], ssA)

            @pl.when(vj == NVJ - 1)
            def _():
                pltpu.async_copy(
                    dsts[sl].at[pl.ds(0, VJ7)],
                    z_hbm.at[s, pl.ds((NVJ - 1) * 16, VJ7), bt], ssB)

        def wait_scatter(j, sl):
            _, vj, _ = task_svb(j)

            @pl.when(vj < NVJ - 1)
            def _():
                pltpu.make_async_copy(
                    dsts[sl].at[pl.ds(0, 16)],
                    z_hbm.at[0, pl.ds(0, 16), 0], ssA).wait()

            @pl.when(vj == NVJ - 1)
            def _():
                pltpu.make_async_copy(
                    dsts[sl].at[pl.ds(0, VJ7)],
                    z_hbm.at[0, pl.ds(0, VJ7), 0], ssB).wait()

        start_gather(0, 0)

        def pair_body(g, carry):
            for b in range(2):
                j = 2 * g + b

                @pl.when(j + 1 < t_per_w)
                def _():
                    start_gather(j + 1, 1 - b)

                wait_gather(b)

                @pl.when(j >= 2)
                def _():
                    wait_scatter(j - 2, b)

                _, vj, _ = task_svb(j)

                @pl.when(vj < NVJ - 1)
                def _():
                    transpose(b, 128)

                @pl.when(vj == NVJ - 1)
                def _():
                    transpose(b, VJ7 * 8)

                start_scatter(j, b)
            return carry

        lax.fori_loop(0, t_per_w // 2, pair_body, 0)
        wait_scatter(t_per_w - 2, 0)
        wait_scatter(t_per_w - 1, 1)

    return tg_kernel


def kernel(contexts, table):
    B, S = contexts.shape
    V, D = table.shape
    DP = (D + 127) // 128 * 128
    table8 = jnp.pad(table, ((0, 0), (0, DP - D))).reshape(V * (DP // 128), 128)
    idx8 = (contexts.astype(jnp.int32).T[:, None, :] * (DP // 128)
            + jnp.arange(DP // 128, dtype=jnp.int32)[None, :, None])
    z = _make_sc_tgather(B, S, D, DP)(table8, idx8.reshape(-1))
    return z.transpose((2, 4, 0, 1, 3)).reshape(B, S, D)
